# Initial kernel scaffold; baseline (speedup 1.0000x reference)
import functools

import jax
import jax.numpy as jnp
from jax import lax
from jax.experimental import pallas as pl
from jax.experimental.pallas import tpu as pltpu
from jax.experimental.pallas import tpu_sc as plsc

N = 10000
E = 320000
D = 128
H = 4
DK = D // H

NC = 2
NS = 16
L = 16
NW = NC * NS
EW = E // NW
C = 80
G = C // L
NCHUNK = EW // C
SRCW = 272
ACCW = 144
NPT = N // NS

_INV_SQRT_DK = 1.0 / (DK ** 0.5)


def _pre_body(x_ref, wsrc_ref, wq_ref, src_ref, q_ref):
    x = x_ref[...]
    src_ref[...] = jax.lax.dot(x, wsrc_ref[...],
                               preferred_element_type=jnp.float32)
    q_ref[...] = jax.lax.dot(x, wq_ref[...],
                             preferred_element_type=jnp.float32)


def _tc_pre(x, wsrc, wq):
    bn = 1000
    grid = (N // bn,)
    return pl.pallas_call(
        _pre_body,
        grid=grid,
        in_specs=[
            pl.BlockSpec((bn, D), lambda i: (i, 0)),
            pl.BlockSpec((D, SRCW), lambda i: (0, 0)),
            pl.BlockSpec((D, D), lambda i: (0, 0)),
        ],
        out_specs=[
            pl.BlockSpec((bn, SRCW), lambda i: (i, 0)),
            pl.BlockSpec((bn, D), lambda i: (i, 0)),
        ],
        out_shape=[
            jax.ShapeDtypeStruct((N, SRCW), jnp.float32),
            jax.ShapeDtypeStruct((N, D), jnp.float32),
        ],
    )(x, wsrc, wq)


def _edge_body(src_tab, q_tab, srci, dsti, zrows, out,
               idx_s, idx_d, fat, qbuf, att, acc, sem_g, sem_q):
    cid = lax.axis_index("c")
    sid = lax.axis_index("s")
    wid = sid * NC + cid

    pltpu.sync_copy(zrows, acc.at[pl.ds(sid * NPT, NPT)])
    pltpu.sync_copy(zrows.at[pl.ds(0, C)], att)
    plsc.subcore_barrier()

    lane = lax.iota(jnp.int32, 16)

    def chunk_body(ci, carry):
        base = wid * EW + ci * C
        pltpu.sync_copy(srci.at[pl.ds(base, C)], idx_s)
        pltpu.sync_copy(dsti.at[pl.ds(base, C)], idx_d)
        cg = pltpu.async_copy(src_tab.at[idx_s], fat, sem_g)
        cq = pltpu.async_copy(q_tab.at[idx_d], qbuf, sem_q)
        cg.wait()
        cq.wait()

        def group_body(g, carry2):
            row = lane + g * L
            scores = []
            for h in range(H):
                s = jnp.zeros((16,), jnp.float32)
                for j in range(DK):
                    col = jnp.full((16,), h * DK + j, jnp.int32)
                    kt = plsc.load_gather(fat, [row, col])
                    qt = plsc.load_gather(qbuf, [row, col])
                    s = s + kt * qt
                a = plsc.load_gather(
                    fat, [row, jnp.full((16,), 2 * D + h, jnp.int32)])
                scores.append(s * _INV_SQRT_DK + a)
            m = jnp.maximum(jnp.maximum(scores[0], scores[1]),
                            jnp.maximum(scores[2], scores[3]))
            exps = [jnp.exp(s - m) for s in scores]
            tot = (exps[0] + exps[1]) + (exps[2] + exps[3])
            attn = [e / tot for e in exps]
            for dd in range(D):
                col = jnp.full((16,), dd, jnp.int32)
                vt = plsc.load_gather(
                    fat, [row, jnp.full((16,), D + dd, jnp.int32)])
                plsc.store_scatter(att, [row, col], attn[dd // DK] * vt)
            plsc.store_scatter(att, [row, jnp.full((16,), D, jnp.int32)],
                               jnp.full((16,), 1.0, jnp.float32))
            return carry2

        lax.fori_loop(0, G, group_body, 0)
        pltpu.sync_copy(att, acc.at[idx_d], add=True)
        return carry

    lax.fori_loop(0, NCHUNK, chunk_body, 0)
    plsc.subcore_barrier()
    pltpu.sync_copy(acc.at[pl.ds(sid * NPT, NPT)],
                    out.at[pl.ds(cid * N + sid * NPT, NPT)])


def _sc_edges(src_tab, q_tab, srci, dsti, zrows):
    mesh = plsc.VectorSubcoreMesh(core_axis_name="c", subcore_axis_name="s")
    fn = functools.partial(
        pl.kernel,
        out_type=jax.ShapeDtypeStruct((NC * N, ACCW), jnp.float32),
        mesh=mesh,
        scratch_types=[
            pltpu.VMEM((C,), jnp.int32),
            pltpu.VMEM((C,), jnp.int32),
            pltpu.VMEM((C, SRCW), jnp.float32),
            pltpu.VMEM((C, D), jnp.float32),
            pltpu.VMEM((C, ACCW), jnp.float32),
            pltpu.VMEM_SHARED((N, ACCW), jnp.float32),
            pltpu.SemaphoreType.DMA,
            pltpu.SemaphoreType.DMA,
        ],
    )(_edge_body)
    return fn(src_tab, q_tab, srci, dsti, zrows)


def _post_body(m0_ref, m1_ref, x_ref, wc_ref, bc_ref, g_ref, b_ref, out_ref):
    m = m0_ref[...] + m1_ref[...]
    msg = m[:, :D]
    w = jnp.maximum(m[:, D:D + 1], 1e-8)
    agg = msg / w
    y = jax.lax.dot(agg, wc_ref[...], preferred_element_type=jnp.float32)
    y = y + bc_ref[...] + x_ref[...]
    mu = jnp.mean(y, axis=-1, keepdims=True)
    var = jnp.mean((y - mu) * (y - mu), axis=-1, keepdims=True)
    out_ref[...] = (y - mu) * jax.lax.rsqrt(var + 1e-5) * g_ref[...] + b_ref[...]


def _tc_post(acc2, x, wc, bc, gamma, beta):
    bn = 1000
    grid = (N // bn,)
    return pl.pallas_call(
        _post_body,
        grid=grid,
        in_specs=[
            pl.BlockSpec((bn, ACCW), lambda i: (i, 0)),
            pl.BlockSpec((bn, ACCW), lambda i: (i + N // bn, 0)),
            pl.BlockSpec((bn, D), lambda i: (i, 0)),
            pl.BlockSpec((D, D), lambda i: (0, 0)),
            pl.BlockSpec((1, D), lambda i: (0, 0)),
            pl.BlockSpec((1, D), lambda i: (0, 0)),
            pl.BlockSpec((1, D), lambda i: (0, 0)),
        ],
        out_specs=pl.BlockSpec((bn, D), lambda i: (i, 0)),
        out_shape=jax.ShapeDtypeStruct((N, D), jnp.float32),
    )(acc2, acc2, x, wc, bc, gamma, beta)


def kernel(x_node, edge_index, Wk, Wq, Wv, Wa, Wmsg, b_msg, Wagg, b_agg,
           gamma, beta):
    wsrc = jnp.concatenate(
        [Wk, Wv, Wk @ Wa, jnp.zeros((D, SRCW - 2 * D - H), jnp.float32)],
        axis=1)
    wc = Wmsg @ Wagg
    bc = (b_msg @ Wagg + b_agg).reshape(1, D)
    srci = edge_index[0]
    dsti = edge_index[1]
    zrows = jnp.zeros((NPT, ACCW), jnp.float32)

    src_tab, q_tab = _tc_pre(x_node, wsrc, Wq)
    acc2 = _sc_edges(src_tab, q_tab, srci, dsti, zrows)
    return _tc_post(acc2, x_node, wc, bc, gamma.reshape(1, D),
                    beta.reshape(1, D))


# speedup vs baseline: 2.1077x; 2.1077x over previous
import functools

import jax
import jax.numpy as jnp
from jax import lax
from jax.experimental import pallas as pl
from jax.experimental.pallas import tpu as pltpu
from jax.experimental.pallas import tpu_sc as plsc

N = 10000
E = 320000
D = 128
H = 4
DK = D // H

NC = 2
NS = 16
L = 16
NW = NC * NS
EW = E // NW
C = 80
G = C // L
NCHUNK = EW // C
SRCW = 272
ACCW = 136
NPAD = 10112
NPT = NPAD // NS

_INV_SQRT_DK = 1.0 / (DK ** 0.5)


def _pre_body(x_ref, wsrc_ref, wq_ref, src_ref, q_ref):
    x = x_ref[...]
    src_ref[...] = jax.lax.dot(x, wsrc_ref[...],
                               preferred_element_type=jnp.float32)
    q_ref[...] = jax.lax.dot(x, wq_ref[...],
                             preferred_element_type=jnp.float32)


def _tc_pre(x, wsrc, wq):
    bn = 1000
    grid = (N // bn,)
    return pl.pallas_call(
        _pre_body,
        grid=grid,
        in_specs=[
            pl.BlockSpec((bn, D), lambda i: (i, 0)),
            pl.BlockSpec((D, SRCW), lambda i: (0, 0)),
            pl.BlockSpec((D, D), lambda i: (0, 0)),
        ],
        out_specs=[
            pl.BlockSpec((bn, SRCW), lambda i: (i, 0)),
            pl.BlockSpec((bn, D), lambda i: (i, 0)),
        ],
        out_shape=[
            jax.ShapeDtypeStruct((N, SRCW), jnp.float32),
            jax.ShapeDtypeStruct((N, D), jnp.float32),
        ],
    )(x, wsrc, wq)


def _edge_body(src_tab, q_tab, srci, dsti, zrows, out,
               idx_s, idx_d, fat, qbuf, att, acc, sem_g, sem_q):
    cid = lax.axis_index("c")
    sid = lax.axis_index("s")
    wid = sid * NC + cid

    pltpu.sync_copy(zrows, acc.at[pl.ds(sid * NPT, NPT)])
    pltpu.sync_copy(zrows.at[pl.ds(0, C)], att)
    plsc.subcore_barrier()

    lane = lax.iota(jnp.int32, 16)

    def chunk_body(ci, carry):
        base = wid * EW + ci * C
        pltpu.sync_copy(srci.at[pl.ds(base, C)], idx_s)
        pltpu.sync_copy(dsti.at[pl.ds(base, C)], idx_d)
        cg = pltpu.async_copy(src_tab.at[idx_s], fat, sem_g)
        cq = pltpu.async_copy(q_tab.at[idx_d], qbuf, sem_q)
        cg.wait()
        cq.wait()

        def group_body(g, carry2):
            row = lane + g * L
            scores = []
            for h in range(H):
                s = jnp.zeros((16,), jnp.float32)
                for j in range(DK):
                    col = jnp.full((16,), h * DK + j, jnp.int32)
                    kt = plsc.load_gather(fat, [row, col])
                    qt = plsc.load_gather(qbuf, [row, col])
                    s = s + kt * qt
                a = plsc.load_gather(
                    fat, [row, jnp.full((16,), 2 * D + h, jnp.int32)])
                scores.append(s * _INV_SQRT_DK + a)
            m = jnp.maximum(jnp.maximum(scores[0], scores[1]),
                            jnp.maximum(scores[2], scores[3]))
            exps = [jnp.exp(s - m) for s in scores]
            tot = (exps[0] + exps[1]) + (exps[2] + exps[3])
            attn = [e / tot for e in exps]
            for dd in range(D):
                col = jnp.full((16,), dd, jnp.int32)
                vt = plsc.load_gather(
                    fat, [row, jnp.full((16,), D + dd, jnp.int32)])
                plsc.store_scatter(att, [row, col], attn[dd // DK] * vt)
            plsc.store_scatter(att, [row, jnp.full((16,), D, jnp.int32)],
                               jnp.full((16,), 1.0, jnp.float32))
            return carry2

        lax.fori_loop(0, G, group_body, 0)
        pltpu.sync_copy(att, acc.at[idx_d], add=True)
        return carry

    lax.fori_loop(0, NCHUNK, chunk_body, 0)
    plsc.subcore_barrier()
    pltpu.sync_copy(acc.at[pl.ds(sid * NPT, NPT)],
                    out.at[cid, pl.ds(sid * NPT, NPT)])


def _sc_edges(src_tab, q_tab, srci, dsti, zrows):
    mesh = plsc.VectorSubcoreMesh(core_axis_name="c", subcore_axis_name="s")
    fn = functools.partial(
        pl.kernel,
        out_type=jax.ShapeDtypeStruct((NC, NPAD, ACCW), jnp.float32),
        mesh=mesh,
        scratch_types=[
            pltpu.VMEM((C,), jnp.int32),
            pltpu.VMEM((C,), jnp.int32),
            pltpu.VMEM((C, SRCW), jnp.float32),
            pltpu.VMEM((C, D), jnp.float32),
            pltpu.VMEM((C, ACCW), jnp.float32),
            pltpu.VMEM_SHARED((NPAD, ACCW), jnp.float32),
            pltpu.SemaphoreType.DMA,
            pltpu.SemaphoreType.DMA,
        ],
        compiler_params=pltpu.CompilerParams(use_tc_tiling_on_sc=False,
                                             needs_layout_passes=False),
    )(_edge_body)
    return fn(src_tab, q_tab, srci, dsti, zrows)


def _post_body(m0_ref, m1_ref, x_ref, wc_ref, bc_ref, g_ref, b_ref, out_ref):
    m = m0_ref[0] + m1_ref[0]
    msg = m[:, :D]
    w = jnp.maximum(m[:, D:D + 1], 1e-8)
    agg = msg / w
    y = jax.lax.dot(agg, wc_ref[...], preferred_element_type=jnp.float32)
    y = y + bc_ref[...] + x_ref[...]
    mu = jnp.mean(y, axis=-1, keepdims=True)
    var = jnp.mean((y - mu) * (y - mu), axis=-1, keepdims=True)
    out_ref[...] = (y - mu) * jax.lax.rsqrt(var + 1e-5) * g_ref[...] + b_ref[...]


def _tc_post(acc2, x, wc, bc, gamma, beta):
    bn = 1000
    grid = (N // bn,)
    return pl.pallas_call(
        _post_body,
        grid=grid,
        in_specs=[
            pl.BlockSpec((1, bn, ACCW), lambda i: (0, i, 0)),
            pl.BlockSpec((1, bn, ACCW), lambda i: (1, i, 0)),
            pl.BlockSpec((bn, D), lambda i: (i, 0)),
            pl.BlockSpec((D, D), lambda i: (0, 0)),
            pl.BlockSpec((1, D), lambda i: (0, 0)),
            pl.BlockSpec((1, D), lambda i: (0, 0)),
            pl.BlockSpec((1, D), lambda i: (0, 0)),
        ],
        out_specs=pl.BlockSpec((bn, D), lambda i: (i, 0)),
        out_shape=jax.ShapeDtypeStruct((N, D), jnp.float32),
    )(acc2, acc2, x, wc, bc, gamma, beta)


def kernel(x_node, edge_index, Wk, Wq, Wv, Wa, Wmsg, b_msg, Wagg, b_agg,
           gamma, beta):
    wsrc = jnp.concatenate(
        [Wk, Wv, Wk @ Wa, jnp.zeros((D, SRCW - 2 * D - H), jnp.float32)],
        axis=1)
    wc = Wmsg @ Wagg
    bc = (b_msg @ Wagg + b_agg).reshape(1, D)
    srci = edge_index[0]
    dsti = edge_index[1]
    zrows = jnp.zeros((NPT, ACCW), jnp.float32)

    src_tab, q_tab = _tc_pre(x_node, wsrc, Wq)
    acc2 = _sc_edges(src_tab, q_tab, srci, dsti, zrows)
    return _tc_post(acc2, x_node, wc, bc, gamma.reshape(1, D),
                    beta.reshape(1, D))


# speedup vs baseline: 2.5158x; 1.1936x over previous
import functools

import jax
import jax.numpy as jnp
from jax import lax
from jax.experimental import pallas as pl
from jax.experimental.pallas import tpu as pltpu
from jax.experimental.pallas import tpu_sc as plsc

N = 10000
E = 320000
D = 128
H = 4
DK = D // H

NC = 2
NS = 16
L = 16
NW = NC * NS
C = 32
G = C // L
CH_W = 320
EWP = C * CH_W
EP = EWP * NW
NP = 10240
SRCW = 264
ACCW = 136
NPAD = 10112
NPT = NPAD // NS
NQUAD = CH_W // 4

_INV_SQRT_DK = 1.0 / (DK ** 0.5)


def _pre_body(x_ref, wsrc_ref, wq_ref, src_ref, q_ref):
    x = x_ref[...]
    src_ref[...] = jax.lax.dot(x, wsrc_ref[...],
                               preferred_element_type=jnp.float32)
    q_ref[...] = jax.lax.dot(x, wq_ref[...],
                             preferred_element_type=jnp.float32)


def _tc_pre(x, wsrc, wq):
    bn = 1024
    grid = (NP // bn,)
    return pl.pallas_call(
        _pre_body,
        grid=grid,
        in_specs=[
            pl.BlockSpec((bn, D), lambda i: (i, 0)),
            pl.BlockSpec((D, SRCW), lambda i: (0, 0)),
            pl.BlockSpec((D, D), lambda i: (0, 0)),
        ],
        out_specs=[
            pl.BlockSpec((bn, SRCW), lambda i: (i, 0)),
            pl.BlockSpec((bn, D), lambda i: (i, 0)),
        ],
        out_shape=[
            jax.ShapeDtypeStruct((NP, SRCW), jnp.float32),
            jax.ShapeDtypeStruct((NP, D), jnp.float32),
        ],
    )(x, wsrc, wq)


def _chunk_compute(fat, qbuf, att):
    lane = lax.iota(jnp.int32, 16)

    def group_body(g, carry):
        row = lane + g * L
        scores = []
        for h in range(H):
            s = jnp.zeros((16,), jnp.float32)
            for j in range(DK):
                col = jnp.full((16,), h * DK + j, jnp.int32)
                kt = plsc.load_gather(fat, [row, col])
                qt = plsc.load_gather(qbuf, [row, col])
                s = s + kt * qt
            a = plsc.load_gather(
                fat, [row, jnp.full((16,), 2 * D + h, jnp.int32)])
            scores.append(s * _INV_SQRT_DK + a)
        m = jnp.maximum(jnp.maximum(scores[0], scores[1]),
                        jnp.maximum(scores[2], scores[3]))
        exps = [jnp.exp(s - m) for s in scores]
        tot = (exps[0] + exps[1]) + (exps[2] + exps[3])
        attn = [e / tot for e in exps]
        for dd in range(D):
            col = jnp.full((16,), dd, jnp.int32)
            vt = plsc.load_gather(
                fat, [row, jnp.full((16,), D + dd, jnp.int32)])
            plsc.store_scatter(att, [row, col], attn[dd // DK] * vt)
        plsc.store_scatter(att, [row, jnp.full((16,), D, jnp.int32)],
                           jnp.full((16,), 1.0, jnp.float32))
        return carry

    lax.fori_loop(0, G, group_body, 0)


def _edge_body(src_tab, q_tab, srci, dsti, zrows, out,
               ids, idd, fat0, fat1, qb0, qb1, att0, att1, acc,
               semg0, semg1, sema0, sema1):
    cid = lax.axis_index("c")
    sid = lax.axis_index("s")
    wid = sid * NC + cid

    pltpu.sync_copy(zrows, acc.at[pl.ds(sid * NPT, NPT)])
    pltpu.sync_copy(zrows.at[pl.ds(0, C)], att0)
    pltpu.sync_copy(zrows.at[pl.ds(0, C)], att1)
    plsc.subcore_barrier()

    ebase = wid * EWP
    pltpu.sync_copy(srci.at[pl.ds(ebase, C)], ids.at[0])
    pltpu.sync_copy(dsti.at[pl.ds(ebase, C)], idd.at[0])
    pltpu.async_copy(att0, acc.at[idd.at[0]], sema0, add=True)
    pltpu.async_copy(att1, acc.at[idd.at[0]], sema1, add=True)
    pltpu.async_copy(src_tab.at[ids.at[0]], fat0, semg0)
    pltpu.async_copy(q_tab.at[idd.at[0]], qb0, semg0)

    bufs = ((fat0, qb0, att0, semg0, sema0),
            (fat1, qb1, att1, semg1, sema1))

    def quad_body(i, carry):
        for j in range(4):
            ch = i * 4 + j
            fatb, qbb, attb, semg, sema = bufs[j % 2]
            fatn, qbn, _, semgn, _ = bufs[(j + 1) % 2]
            sj = (j + 1) % 4

            def prefetch():
                off = ebase + (ch + 1) * C
                pltpu.sync_copy(srci.at[pl.ds(off, C)], ids.at[sj])
                pltpu.sync_copy(dsti.at[pl.ds(off, C)], idd.at[sj])
                pltpu.async_copy(src_tab.at[ids.at[sj]], fatn, semgn)
                pltpu.async_copy(q_tab.at[idd.at[sj]], qbn, semgn)

            if j < 3:
                prefetch()
            else:
                @pl.when(i < NQUAD - 1)
                def _():
                    prefetch()
            pltpu.make_async_copy(src_tab.at[pl.ds(0, C)], fatb, semg).wait()
            pltpu.make_async_copy(q_tab.at[pl.ds(0, C)], qbb, semg).wait()
            pltpu.make_async_copy(zrows.at[pl.ds(0, C)], attb, sema).wait()
            _chunk_compute(fatb, qbb, attb)
            pltpu.async_copy(attb, acc.at[idd.at[j]], sema, add=True)
        return carry

    lax.fori_loop(0, NQUAD, quad_body, 0)
    pltpu.make_async_copy(zrows.at[pl.ds(0, C)], att0, sema0).wait()
    pltpu.make_async_copy(zrows.at[pl.ds(0, C)], att1, sema1).wait()
    plsc.subcore_barrier()
    pltpu.sync_copy(acc.at[pl.ds(sid * NPT, NPT)],
                    out.at[cid, pl.ds(sid * NPT, NPT)])


def _sc_edges(src_tab, q_tab, srci, dsti, zrows):
    mesh = plsc.VectorSubcoreMesh(core_axis_name="c", subcore_axis_name="s")
    fn = functools.partial(
        pl.kernel,
        out_type=jax.ShapeDtypeStruct((NC, NPAD, ACCW), jnp.float32),
        mesh=mesh,
        scratch_types=[
            pltpu.VMEM((4, C), jnp.int32),
            pltpu.VMEM((4, C), jnp.int32),
            pltpu.VMEM((C, SRCW), jnp.float32),
            pltpu.VMEM((C, SRCW), jnp.float32),
            pltpu.VMEM((C, D), jnp.float32),
            pltpu.VMEM((C, D), jnp.float32),
            pltpu.VMEM((C, ACCW), jnp.float32),
            pltpu.VMEM((C, ACCW), jnp.float32),
            pltpu.VMEM_SHARED((NPAD, ACCW), jnp.float32),
            pltpu.SemaphoreType.DMA,
            pltpu.SemaphoreType.DMA,
            pltpu.SemaphoreType.DMA,
            pltpu.SemaphoreType.DMA,
        ],
        compiler_params=pltpu.CompilerParams(use_tc_tiling_on_sc=False,
                                             needs_layout_passes=False),
    )(_edge_body)
    return fn(src_tab, q_tab, srci, dsti, zrows)


def _post_body(m0_ref, m1_ref, x_ref, wc_ref, bc_ref, g_ref, b_ref, out_ref):
    m = m0_ref[0] + m1_ref[0]
    msg = m[:, :D]
    w = jnp.maximum(m[:, D:D + 1], 1e-8)
    agg = msg / w
    y = jax.lax.dot(agg, wc_ref[...], preferred_element_type=jnp.float32)
    y = y + bc_ref[...] + x_ref[...]
    mu = jnp.mean(y, axis=-1, keepdims=True)
    var = jnp.mean((y - mu) * (y - mu), axis=-1, keepdims=True)
    out_ref[...] = (y - mu) * jax.lax.rsqrt(var + 1e-5) * g_ref[...] + b_ref[...]


def _tc_post(acc2, x, wc, bc, gamma, beta):
    bn = 1000
    grid = (N // bn,)
    return pl.pallas_call(
        _post_body,
        grid=grid,
        in_specs=[
            pl.BlockSpec((1, bn, ACCW), lambda i: (0, i, 0)),
            pl.BlockSpec((1, bn, ACCW), lambda i: (1, i, 0)),
            pl.BlockSpec((bn, D), lambda i: (i, 0)),
            pl.BlockSpec((D, D), lambda i: (0, 0)),
            pl.BlockSpec((1, D), lambda i: (0, 0)),
            pl.BlockSpec((1, D), lambda i: (0, 0)),
            pl.BlockSpec((1, D), lambda i: (0, 0)),
        ],
        out_specs=pl.BlockSpec((bn, D), lambda i: (i, 0)),
        out_shape=jax.ShapeDtypeStruct((N, D), jnp.float32),
    )(acc2, acc2, x, wc, bc, gamma, beta)


def kernel(x_node, edge_index, Wk, Wq, Wv, Wa, Wmsg, b_msg, Wagg, b_agg,
           gamma, beta):
    wsrc = jnp.concatenate(
        [Wk, Wv, Wk @ Wa, jnp.zeros((D, SRCW - 2 * D - H), jnp.float32)],
        axis=1)
    wc = Wmsg @ Wagg
    bc = (b_msg @ Wagg + b_agg).reshape(1, D)
    pad = EP - E
    srci = jnp.concatenate([edge_index[0], jnp.zeros((pad,), jnp.int32)])
    dsti = jnp.concatenate([edge_index[1], jnp.full((pad,), N, jnp.int32)])
    xp = jnp.concatenate([x_node, jnp.zeros((NP - N, D), jnp.float32)])
    zrows = jnp.zeros((NPT, ACCW), jnp.float32)

    src_tab, q_tab = _tc_pre(xp, wsrc, Wq)
    acc2 = _sc_edges(src_tab, q_tab, srci, dsti, zrows)
    return _tc_post(acc2, x_node, wc, bc, gamma.reshape(1, D),
                    beta.reshape(1, D))


# speedup vs baseline: 2.9430x; 1.1698x over previous
import functools

import jax
import jax.numpy as jnp
from jax import lax
from jax.experimental import pallas as pl
from jax.experimental.pallas import tpu as pltpu
from jax.experimental.pallas import tpu_sc as plsc

N = 10000
E = 320000
D = 128
H = 4
DK = D // H

NC = 2
NS = 16
L = 16
NW = NC * NS
C = 32
G = C // L
CH_W = 320
EWP = C * CH_W
EP = EWP * NW
NP = 10240
SRCW = 264
ACCW = 136
NPAD = 10112
NPT = NPAD // NS
NQUAD = CH_W // 4

_INV_SQRT_DK = 1.0 / (DK ** 0.5)


def _pre_body(x_ref, wsrc_ref, wq_ref, src_ref, q_ref):
    x = x_ref[...]
    src_ref[...] = jax.lax.dot(x, wsrc_ref[...],
                               preferred_element_type=jnp.float32)
    q_ref[...] = jax.lax.dot(x, wq_ref[...],
                             preferred_element_type=jnp.float32)


def _tc_pre(x, wsrc, wq):
    bn = 1024
    grid = (NP // bn,)
    return pl.pallas_call(
        _pre_body,
        grid=grid,
        in_specs=[
            pl.BlockSpec((bn, D), lambda i: (i, 0)),
            pl.BlockSpec((D, SRCW), lambda i: (0, 0)),
            pl.BlockSpec((D, D), lambda i: (0, 0)),
        ],
        out_specs=[
            pl.BlockSpec((bn, SRCW), lambda i: (i, 0)),
            pl.BlockSpec((bn, D), lambda i: (i, 0)),
        ],
        out_shape=[
            jax.ShapeDtypeStruct((NP, SRCW), jnp.float32),
            jax.ShapeDtypeStruct((NP, D), jnp.float32),
        ],
    )(x, wsrc, wq)


def _chunk_compute(fat, qbuf, att):
    lane = lax.iota(jnp.int32, 16)

    def group_body(g, carry):
        row = lane + g * L
        scores = []
        for h in range(H):
            s = jnp.zeros((16,), jnp.float32)
            for j in range(DK):
                col = jnp.full((16,), h * DK + j, jnp.int32)
                kt = plsc.load_gather(fat, [row, col])
                qt = plsc.load_gather(qbuf, [row, col])
                s = s + kt * qt
            a = plsc.load_gather(
                fat, [row, jnp.full((16,), 2 * D + h, jnp.int32)])
            scores.append(s * _INV_SQRT_DK + a)
        m = jnp.maximum(jnp.maximum(scores[0], scores[1]),
                        jnp.maximum(scores[2], scores[3]))
        exps = [jnp.exp(s - m) for s in scores]
        tot = (exps[0] + exps[1]) + (exps[2] + exps[3])
        attn = [e / tot for e in exps]
        for dd in range(D):
            col = jnp.full((16,), dd, jnp.int32)
            vt = plsc.load_gather(
                fat, [row, jnp.full((16,), D + dd, jnp.int32)])
            plsc.store_scatter(att, [row, col], attn[dd // DK] * vt)
        plsc.store_scatter(att, [row, jnp.full((16,), D, jnp.int32)],
                           jnp.full((16,), 1.0, jnp.float32))
        return carry

    lax.fori_loop(0, G, group_body, 0)


def _edge_body(src_tab, q_tab, srci, dsti, zrows, out,
               ids, idd, fat0, fat1, qb0, qb1, att0, att1, acc,
               semg0, semg1, sema0, sema1, semi0, semi1):
    cid = lax.axis_index("c")
    sid = lax.axis_index("s")
    wid = sid * NC + cid

    pltpu.sync_copy(zrows, acc.at[pl.ds(sid * NPT, NPT)])
    pltpu.sync_copy(zrows.at[pl.ds(0, C)], att0)
    pltpu.sync_copy(zrows.at[pl.ds(0, C)], att1)
    plsc.subcore_barrier()

    ebase = wid * EWP
    pltpu.sync_copy(srci.at[pl.ds(ebase, C)], ids.at[0])
    pltpu.sync_copy(dsti.at[pl.ds(ebase, C)], idd.at[0])
    pltpu.async_copy(srci.at[pl.ds(ebase + C, C)], ids.at[1], semi1)
    pltpu.async_copy(dsti.at[pl.ds(ebase + C, C)], idd.at[1], semi1)
    pltpu.async_copy(att0, acc.at[idd.at[0]], sema0, add=True)
    pltpu.async_copy(att1, acc.at[idd.at[0]], sema1, add=True)
    pltpu.async_copy(src_tab.at[ids.at[0]], fat0, semg0)
    pltpu.async_copy(q_tab.at[idd.at[0]], qb0, semg0)

    bufs = ((fat0, qb0, att0, semg0, sema0),
            (fat1, qb1, att1, semg1, sema1))
    semi = (semi0, semi1)

    def quad_body(i, carry):
        for j in range(4):
            ch = i * 4 + j
            fatb, qbb, attb, semg, sema = bufs[j % 2]
            fatn, qbn, _, semgn, _ = bufs[(j + 1) % 2]

            pltpu.make_async_copy(zrows.at[pl.ds(0, C)], attb, sema).wait()

            def idx_issue():
                off = ebase + (ch + 2) * C
                pltpu.async_copy(srci.at[pl.ds(off, C)],
                                 ids.at[(j + 2) % 4], semi[j % 2])
                pltpu.async_copy(dsti.at[pl.ds(off, C)],
                                 idd.at[(j + 2) % 4], semi[j % 2])
            if j < 2:
                idx_issue()
            else:
                @pl.when(i < NQUAD - 1)
                def _():
                    idx_issue()

            def nxt_gather():
                sj = (j + 1) % 4
                pltpu.make_async_copy(srci.at[pl.ds(0, C)],
                                      ids.at[sj], semi[(j + 1) % 2]).wait()
                pltpu.make_async_copy(dsti.at[pl.ds(0, C)],
                                      idd.at[sj], semi[(j + 1) % 2]).wait()
                pltpu.async_copy(src_tab.at[ids.at[sj]], fatn, semgn)
                pltpu.async_copy(q_tab.at[idd.at[sj]], qbn, semgn)
            if j < 3:
                nxt_gather()
            else:
                @pl.when(i < NQUAD - 1)
                def _():
                    nxt_gather()

            pltpu.make_async_copy(src_tab.at[pl.ds(0, C)], fatb, semg).wait()
            pltpu.make_async_copy(q_tab.at[pl.ds(0, C)], qbb, semg).wait()
            _chunk_compute(fatb, qbb, attb)
            pltpu.async_copy(attb, acc.at[idd.at[j]], sema, add=True)
        return carry

    lax.fori_loop(0, NQUAD, quad_body, 0)
    pltpu.make_async_copy(zrows.at[pl.ds(0, C)], att0, sema0).wait()
    pltpu.make_async_copy(zrows.at[pl.ds(0, C)], att1, sema1).wait()
    plsc.subcore_barrier()
    pltpu.sync_copy(acc.at[pl.ds(sid * NPT, NPT)],
                    out.at[cid, pl.ds(sid * NPT, NPT)])


def _sc_edges(src_tab, q_tab, srci, dsti, zrows):
    mesh = plsc.VectorSubcoreMesh(core_axis_name="c", subcore_axis_name="s")
    fn = functools.partial(
        pl.kernel,
        out_type=jax.ShapeDtypeStruct((NC, NPAD, ACCW), jnp.float32),
        mesh=mesh,
        scratch_types=[
            pltpu.VMEM((4, C), jnp.int32),
            pltpu.VMEM((4, C), jnp.int32),
            pltpu.VMEM((C, SRCW), jnp.float32),
            pltpu.VMEM((C, SRCW), jnp.float32),
            pltpu.VMEM((C, D), jnp.float32),
            pltpu.VMEM((C, D), jnp.float32),
            pltpu.VMEM((C, ACCW), jnp.float32),
            pltpu.VMEM((C, ACCW), jnp.float32),
            pltpu.VMEM_SHARED((NPAD, ACCW), jnp.float32),
            pltpu.SemaphoreType.DMA,
            pltpu.SemaphoreType.DMA,
            pltpu.SemaphoreType.DMA,
            pltpu.SemaphoreType.DMA,
            pltpu.SemaphoreType.DMA,
            pltpu.SemaphoreType.DMA,
        ],
        compiler_params=pltpu.CompilerParams(use_tc_tiling_on_sc=False,
                                             needs_layout_passes=False),
    )(_edge_body)
    return fn(src_tab, q_tab, srci, dsti, zrows)


def _post_body(m0_ref, m1_ref, x_ref, wc_ref, bc_ref, g_ref, b_ref, out_ref):
    m = m0_ref[0] + m1_ref[0]
    msg = m[:, :D]
    w = jnp.maximum(m[:, D:D + 1], 1e-8)
    agg = msg / w
    y = jax.lax.dot(agg, wc_ref[...], preferred_element_type=jnp.float32)
    y = y + bc_ref[...] + x_ref[...]
    mu = jnp.mean(y, axis=-1, keepdims=True)
    var = jnp.mean((y - mu) * (y - mu), axis=-1, keepdims=True)
    out_ref[...] = (y - mu) * jax.lax.rsqrt(var + 1e-5) * g_ref[...] + b_ref[...]


def _tc_post(acc2, x, wc, bc, gamma, beta):
    bn = 1000
    grid = (N // bn,)
    return pl.pallas_call(
        _post_body,
        grid=grid,
        in_specs=[
            pl.BlockSpec((1, bn, ACCW), lambda i: (0, i, 0)),
            pl.BlockSpec((1, bn, ACCW), lambda i: (1, i, 0)),
            pl.BlockSpec((bn, D), lambda i: (i, 0)),
            pl.BlockSpec((D, D), lambda i: (0, 0)),
            pl.BlockSpec((1, D), lambda i: (0, 0)),
            pl.BlockSpec((1, D), lambda i: (0, 0)),
            pl.BlockSpec((1, D), lambda i: (0, 0)),
        ],
        out_specs=pl.BlockSpec((bn, D), lambda i: (i, 0)),
        out_shape=jax.ShapeDtypeStruct((N, D), jnp.float32),
    )(acc2, acc2, x, wc, bc, gamma, beta)


def kernel(x_node, edge_index, Wk, Wq, Wv, Wa, Wmsg, b_msg, Wagg, b_agg,
           gamma, beta):
    wsrc = jnp.concatenate(
        [Wk, Wv, Wk @ Wa, jnp.zeros((D, SRCW - 2 * D - H), jnp.float32)],
        axis=1)
    wc = Wmsg @ Wagg
    bc = (b_msg @ Wagg + b_agg).reshape(1, D)
    pad = EP - E
    ar = jnp.arange(pad, dtype=jnp.int32)
    srci = jnp.concatenate([edge_index[0], ar % N])
    dsti = jnp.concatenate([edge_index[1], N + ar % (NPAD - N)])
    xp = jnp.concatenate([x_node, jnp.zeros((NP - N, D), jnp.float32)])
    zrows = jnp.zeros((NPT, ACCW), jnp.float32)

    src_tab, q_tab = _tc_pre(xp, wsrc, Wq)
    acc2 = _sc_edges(src_tab, q_tab, srci, dsti, zrows)
    return _tc_post(acc2, x_node, wc, bc, gamma.reshape(1, D),
                    beta.reshape(1, D))
